# baseline (speedup 1.0000x reference)
import functools

import jax
import jax.numpy as jnp
from jax import lax
from jax.experimental import pallas as pl
from jax.experimental.pallas import tpu as pltpu
from jax.experimental.pallas import tpu_sc as plsc

EMBED = 32
LANES = 16
CHUNK = 128


@functools.cache
def _build_sc(batch):
    info = plsc.get_sparse_core_info()
    nw = info.num_cores * info.num_subcores
    bpw = batch // nw
    nchunk = bpw // CHUNK
    mesh = plsc.VectorSubcoreMesh(core_axis_name="c", subcore_axis_name="s")

    @functools.partial(
        pl.kernel,
        mesh=mesh,
        compiler_params=pltpu.CompilerParams(use_tc_tiling_on_sc=False),
        out_type=jax.ShapeDtypeStruct((batch,), jnp.float32),
        scratch_types=[
            pltpu.VMEM((bpw,), jnp.int32),
            pltpu.VMEM((bpw,), jnp.int32),
            pltpu.VMEM((EMBED, nchunk, CHUNK), jnp.int32),
            pltpu.VMEM((EMBED, nchunk, CHUNK), jnp.int32),
            pltpu.VMEM((EMBED * bpw,), jnp.float32),
            pltpu.VMEM((EMBED * bpw,), jnp.float32),
            pltpu.VMEM((bpw,), jnp.float32),
            pltpu.SemaphoreType.DMA,
        ],
    )
    def bandit(prod_hbm, user_hbm, ptab_hbm, utab_hbm, out_hbm,
               pidx_v, uidx_v, aidx_v, bidx_v, at_v, bt_v, outv, sem):
        wid = lax.axis_index("s") * info.num_cores + lax.axis_index("c")
        base = wid * bpw
        pltpu.sync_copy(prod_hbm.at[wid], pidx_v)
        pltpu.sync_copy(user_hbm.at[wid], uidx_v)

        def expand(d, carry):
            for c in range(nchunk):
                for v in range(CHUNK // LANES):
                    sl = pl.ds(v * LANES, LANES)
                    src = pl.ds(c * CHUNK + v * LANES, LANES)
                    aidx_v[d, c, sl] = pidx_v[src] * EMBED + d
                    bidx_v[d, c, sl] = uidx_v[src] * EMBED + d
            return carry

        lax.fori_loop(0, EMBED, expand, 0)

        handles = []
        for d in range(EMBED):
            for c in range(nchunk):
                dst = pl.ds(d * bpw + c * CHUNK, CHUNK)
                handles.append(pltpu.async_copy(
                    ptab_hbm.at[aidx_v.at[d, c]], at_v.at[dst], sem))
                handles.append(pltpu.async_copy(
                    utab_hbm.at[bidx_v.at[d, c]], bt_v.at[dst], sem))
        for h in handles:
            h.wait()

        def group(g, carry):
            acc = jnp.zeros((LANES,), jnp.float32)
            for d in range(EMBED):
                sl = pl.ds(d * bpw + g * LANES, LANES)
                acc = acc + at_v[sl] * bt_v[sl]
            outv[pl.ds(g * LANES, LANES)] = acc
            return carry

        lax.fori_loop(0, bpw // LANES, group, 0)
        pltpu.sync_copy(outv, out_hbm.at[pl.ds(base, bpw)])

    return bandit


def kernel(products, users, product_embedding, user_embedding):
    batch = products.shape[0]
    nrows, embed = product_embedding.shape
    info = plsc.get_sparse_core_info()
    nw = info.num_cores * info.num_subcores
    prod2 = products.astype(jnp.int32).reshape(nw, batch // nw)
    user2 = users.astype(jnp.int32).reshape(nw, batch // nw)
    ptf = product_embedding.reshape(nrows * embed)
    utf = user_embedding.reshape(nrows * embed)
    return _build_sc(batch)(prod2, user2, ptf, utf)
